# baseline (speedup 1.0000x reference)
import functools

import jax
import jax.numpy as jnp
from jax import lax
from jax.experimental import pallas as pl
from jax.experimental.pallas import tpu as pltpu
from jax.experimental.pallas import tpu_sc as plsc

_N = 10000
_E = 320000
_DIN = 128
_DE = 16
_DOUT = 128
_DTD = 24

_NC = 2
_NS = 16
_NW = _NC * _NS
_EPW = _E // _NW
_K = 80
_C = _EPW // _K
_NP = 10240
_RPT = _NP // _NS
_G = 25
_NG = _C // _G


def _sc_segment_sums(nf, ef, src3, dst3):
    mesh = plsc.VectorSubcoreMesh(core_axis_name="c", subcore_axis_name="s")

    @functools.partial(
        pl.kernel,
        mesh=mesh,
        out_type=[
            jax.ShapeDtypeStruct((_NC, _NP, _DIN), jnp.float32),
            jax.ShapeDtypeStruct((_NC, _NP, _DTD), jnp.float32),
        ],
        scratch_types=[
            pltpu.VMEM_SHARED((_NP, _DIN), jnp.float32),
            pltpu.VMEM_SHARED((_NP, _DTD), jnp.float32),
            pltpu.VMEM((_G, _K), jnp.int32),
            pltpu.VMEM((_G, _K), jnp.int32),
            pltpu.VMEM((2, _K, _DIN), jnp.float32),
            pltpu.VMEM((2, _K, _DTD), jnp.float32),
            pltpu.SemaphoreType.DMA,
            pltpu.SemaphoreType.DMA,
            pltpu.SemaphoreType.DMA,
            pltpu.SemaphoreType.DMA,
        ],
        compiler_params=pltpu.CompilerParams(use_tc_tiling_on_sc=False),
    )
    def k(nf_hbm, ef_hbm, src_hbm, dst_hbm, s_out, td_out,
          s_sp, td_sp, isrc, idst, rows, tdv, sem_g, sem_e, sem_s, sem_td):
        c = lax.axis_index("c")
        s = lax.axis_index("s")
        w = c * _NS + s
        base = s * _RPT
        ebase = w * _EPW

        def zrow(i, carry):
            for h in range(_DIN // 16):
                rows[0, i, pl.ds(h * 16, 16)] = jnp.zeros((16,), jnp.float32)
            tdv[0, i, pl.ds(0, 16)] = jnp.zeros((16,), jnp.float32)
            tdv[0, i, pl.ds(8, 16)] = jnp.zeros((16,), jnp.float32)
            return carry
        lax.fori_loop(0, _K, zrow, 0)

        for z in range(_RPT // _K):
            r0 = base + z * _K
            pltpu.sync_copy(rows.at[0], s_sp.at[pl.ds(r0, _K)])
            pltpu.sync_copy(tdv.at[0], td_sp.at[pl.ds(r0, _K)])

        def orow(i, carry):
            tdv[0, i, pl.ds(8, 16)] = jnp.ones((16,), jnp.float32)
            tdv[1, i, pl.ds(8, 16)] = jnp.ones((16,), jnp.float32)
            return carry
        lax.fori_loop(0, _K, orow, 0)
        plsc.subcore_barrier()

        def stage(g, carry):
            pltpu.sync_copy(src_hbm.at[w, pl.ds(g * _G, _G)], isrc)
            pltpu.sync_copy(dst_hbm.at[w, pl.ds(g * _G, _G)], idst)
            pltpu.async_copy(nf_hbm.at[isrc.at[0]], rows.at[0], sem_g)
            pltpu.async_copy(ef_hbm.at[pl.ds(ebase + g * _G * _K, _K)],
                             tdv.at[0, :, pl.ds(0, _DE)], sem_e)

            def step(j, carry2):
                p = lax.rem(j, 2)
                pltpu.make_async_copy(
                    nf_hbm.at[pl.ds(0, _K)], rows.at[p], sem_g).wait()
                pltpu.make_async_copy(
                    ef_hbm.at[pl.ds(0, _K)],
                    tdv.at[p, :, pl.ds(0, _DE)], sem_e).wait()

                @pl.when(j >= 1)
                def _drain_prev():
                    pltpu.make_async_copy(
                        nf_hbm.at[pl.ds(0, _K)], rows.at[1 - p], sem_s).wait()
                    pltpu.make_async_copy(
                        td_out.at[0, pl.ds(0, _K)], tdv.at[1 - p],
                        sem_td).wait()

                @pl.when(j + 1 < _G)
                def _prefetch():
                    pltpu.async_copy(
                        nf_hbm.at[isrc.at[j + 1]], rows.at[1 - p], sem_g)
                    pltpu.async_copy(
                        ef_hbm.at[pl.ds(ebase + (g * _G + j + 1) * _K, _K)],
                        tdv.at[1 - p, :, pl.ds(0, _DE)], sem_e)

                pltpu.async_copy(rows.at[p], s_sp.at[idst.at[j]], sem_s,
                                 add=True)
                pltpu.async_copy(tdv.at[p], td_sp.at[idst.at[j]], sem_td,
                                 add=True)
                return carry2
            lax.fori_loop(0, _G, step, 0)

            pltpu.make_async_copy(
                nf_hbm.at[pl.ds(0, _K)], rows.at[0], sem_s).wait()
            pltpu.make_async_copy(
                td_out.at[0, pl.ds(0, _K)], tdv.at[0], sem_td).wait()
            return carry
        lax.fori_loop(0, _NG, stage, 0)

        plsc.subcore_barrier()

        pltpu.sync_copy(s_sp.at[pl.ds(base, _RPT)],
                        s_out.at[c, pl.ds(base, _RPT)])
        pltpu.sync_copy(td_sp.at[pl.ds(base, _RPT)],
                        td_out.at[c, pl.ds(base, _RPT)])

    return k(nf, ef, src3, dst3)


def _tc_ef_transpose(eft):
    bw = 2560
    grid = (_E // bw,)

    def body(inr, outr):
        outr[...] = inr[...].T

    return pl.pallas_call(
        body,
        grid=grid,
        in_specs=[pl.BlockSpec((_DE, bw), lambda i: (0, i))],
        out_specs=pl.BlockSpec((bw, _DE), lambda i: (i, 0)),
        out_shape=jax.ShapeDtypeStruct((_E, _DE), jnp.float32),
    )(eft)


def _tc_apply(s2, td2, nf, wma, wmb, bm, wa1, wa2, ba):
    bn = 400
    grid = (_N // bn,)

    def body(s2r, td2r, nfr, wmar, wmbr, bmr, wa1r, wa2r, bar, outr):
        sv = s2r[0] + s2r[1]
        tv = td2r[0, :, 0:_DE] + td2r[1, :, 0:_DE]
        dv = td2r[0, :, _DE:_DE + 1] + td2r[1, :, _DE:_DE + 1]
        sums = (jnp.dot(sv, wmar[...], preferred_element_type=jnp.float32)
                + jnp.dot(tv, wmbr[...], preferred_element_type=jnp.float32)
                + dv * bmr[...])
        hn = sums / jnp.maximum(dv, 1.0)
        out = (jnp.dot(nfr[...], wa1r[...], preferred_element_type=jnp.float32)
               + jnp.dot(hn, wa2r[...], preferred_element_type=jnp.float32)
               + bar[...])
        outr[...] = jnp.maximum(out, 0.0)

    return pl.pallas_call(
        body,
        grid=grid,
        in_specs=[
            pl.BlockSpec((_NC, bn, _DIN), lambda i: (0, i, 0)),
            pl.BlockSpec((_NC, bn, _DTD), lambda i: (0, i, 0)),
            pl.BlockSpec((bn, _DIN), lambda i: (i, 0)),
            pl.BlockSpec((_DIN, _DOUT), lambda i: (0, 0)),
            pl.BlockSpec((_DE, _DOUT), lambda i: (0, 0)),
            pl.BlockSpec((1, _DOUT), lambda i: (0, 0)),
            pl.BlockSpec((_DIN, _DOUT), lambda i: (0, 0)),
            pl.BlockSpec((_DOUT, _DOUT), lambda i: (0, 0)),
            pl.BlockSpec((1, _DOUT), lambda i: (0, 0)),
        ],
        out_specs=pl.BlockSpec((bn, _DOUT), lambda i: (i, 0)),
        out_shape=jax.ShapeDtypeStruct((_N, _DOUT), jnp.float32),
    )(s2, td2, nf, wma, wmb, bm, wa1, wa2, ba)


def kernel(nfeats, efeats, edge_index, W_msg, b_msg, W_apply, b_apply):
    nf = nfeats.reshape(_N, _DIN)
    ef = _tc_ef_transpose(jnp.transpose(efeats, (1, 2, 0)).reshape(_DE, _E))
    src3 = edge_index[0].reshape(_NW, _C, _K)
    dst3 = edge_index[1].reshape(_NW, _C, _K)
    s2, td2 = _sc_segment_sums(nf, ef, src3, dst3)
    wma = W_msg[:, :_DIN].T
    wmb = W_msg[:, _DIN:].T
    wa1 = W_apply[:, :_DIN].T
    wa2 = W_apply[:, _DIN:].T
    out = _tc_apply(s2, td2, nf, wma, wmb,
                    b_msg.reshape(1, _DOUT), wa1, wa2,
                    b_apply.reshape(1, _DOUT))
    return out.reshape(_N, 1, _DOUT)


# speedup vs baseline: 1.3072x; 1.3072x over previous
import functools

import jax
import jax.numpy as jnp
from jax import lax
from jax.experimental import pallas as pl
from jax.experimental.pallas import tpu as pltpu
from jax.experimental.pallas import tpu_sc as plsc

_N = 10000
_E = 320000
_DIN = 128
_DE = 16
_DOUT = 128
_DTD = 24

_NC = 2
_NS = 16
_NW = _NC * _NS
_EPW = _E // _NW
_K = 80
_C = _EPW // _K
_NP = 10240
_RPT = _NP // _NS
_G = 25
_NG = _C // _G


def _sc_segment_sums(nf, ef, src3, dst3):
    mesh = plsc.VectorSubcoreMesh(core_axis_name="c", subcore_axis_name="s")

    @functools.partial(
        pl.kernel,
        mesh=mesh,
        out_type=[
            jax.ShapeDtypeStruct((_NC, _NP, _DIN), jnp.float32),
            jax.ShapeDtypeStruct((_NC, _NP, _DTD), jnp.float32),
        ],
        scratch_types=[
            pltpu.VMEM_SHARED((_NP, _DIN), jnp.float32),
            pltpu.VMEM_SHARED((_NP, _DTD), jnp.float32),
            pltpu.VMEM((_G, _K), jnp.int32),
            pltpu.VMEM((_G, _K), jnp.int32),
            pltpu.VMEM((2, _K, _DIN), jnp.float32),
            pltpu.VMEM((2, _K, _DTD), jnp.float32),
            pltpu.SemaphoreType.DMA,
            pltpu.SemaphoreType.DMA,
            pltpu.SemaphoreType.DMA,
            pltpu.SemaphoreType.DMA,
        ],
        compiler_params=pltpu.CompilerParams(use_tc_tiling_on_sc=False),
    )
    def k(nf_hbm, ef_hbm, src_hbm, dst_hbm, s_out, td_out,
          s_sp, td_sp, isrc, idst, rows, tdv, sem_g, sem_e, sem_s, sem_td):
        c = lax.axis_index("c")
        s = lax.axis_index("s")
        w = c * _NS + s
        base = s * _RPT
        ebase = w * _EPW

        def zrow(i, carry):
            for h in range(_DIN // 16):
                rows[0, i, pl.ds(h * 16, 16)] = jnp.zeros((16,), jnp.float32)
            tdv[0, i, pl.ds(0, 16)] = jnp.zeros((16,), jnp.float32)
            tdv[0, i, pl.ds(8, 16)] = jnp.zeros((16,), jnp.float32)
            return carry
        lax.fori_loop(0, _K, zrow, 0)

        for z in range(_RPT // _K):
            r0 = base + z * _K
            pltpu.sync_copy(rows.at[0], s_sp.at[pl.ds(r0, _K)])
            pltpu.sync_copy(tdv.at[0], td_sp.at[pl.ds(r0, _K)])

        def orow(i, carry):
            tdv[0, i, pl.ds(8, 16)] = jnp.ones((16,), jnp.float32)
            tdv[1, i, pl.ds(8, 16)] = jnp.ones((16,), jnp.float32)
            return carry
        lax.fori_loop(0, _K, orow, 0)
        plsc.subcore_barrier()

        def stage(g, carry):
            pltpu.sync_copy(src_hbm.at[w, pl.ds(g * _G, _G)], isrc)
            pltpu.sync_copy(dst_hbm.at[w, pl.ds(g * _G, _G)], idst)
            pltpu.async_copy(nf_hbm.at[isrc.at[0]], rows.at[0], sem_g)
            pltpu.async_copy(ef_hbm.at[pl.ds(ebase + g * _G * _K, _K)],
                             tdv.at[0, :, pl.ds(0, _DE)], sem_e)

            def step(j, carry2):
                p = lax.rem(j, 2)
                pltpu.make_async_copy(
                    nf_hbm.at[pl.ds(0, _K)], rows.at[p], sem_g).wait()
                pltpu.make_async_copy(
                    ef_hbm.at[pl.ds(0, _K)],
                    tdv.at[p, :, pl.ds(0, _DE)], sem_e).wait()

                @pl.when(j >= 1)
                def _drain_prev():
                    pltpu.make_async_copy(
                        nf_hbm.at[pl.ds(0, _K)], rows.at[1 - p], sem_s).wait()
                    pltpu.make_async_copy(
                        td_out.at[0, pl.ds(0, _K)], tdv.at[1 - p],
                        sem_td).wait()

                @pl.when(j + 1 < _G)
                def _prefetch():
                    pltpu.async_copy(
                        nf_hbm.at[isrc.at[j + 1]], rows.at[1 - p], sem_g)
                    pltpu.async_copy(
                        ef_hbm.at[pl.ds(ebase + (g * _G + j + 1) * _K, _K)],
                        tdv.at[1 - p, :, pl.ds(0, _DE)], sem_e)

                pltpu.async_copy(rows.at[p], s_sp.at[idst.at[j]], sem_s,
                                 add=True)
                pltpu.async_copy(tdv.at[p], td_sp.at[idst.at[j]], sem_td,
                                 add=True)
                return carry2
            lax.fori_loop(0, _G, step, 0)

            pltpu.make_async_copy(
                nf_hbm.at[pl.ds(0, _K)], rows.at[0], sem_s).wait()
            pltpu.make_async_copy(
                td_out.at[0, pl.ds(0, _K)], tdv.at[0], sem_td).wait()
            return carry
        lax.fori_loop(0, _NG, stage, 0)

        plsc.subcore_barrier()

        pltpu.sync_copy(s_sp.at[pl.ds(base, _RPT)],
                        s_out.at[c, pl.ds(base, _RPT)])
        pltpu.sync_copy(td_sp.at[pl.ds(base, _RPT)],
                        td_out.at[c, pl.ds(base, _RPT)])

    return k(nf, ef, src3, dst3)


def _tc_apply(s2, td2, nf, wma, wmb, bm, wa1, wa2, ba):
    bn = 400
    grid = (_N // bn,)

    def body(s2r, td2r, nfr, wmar, wmbr, bmr, wa1r, wa2r, bar, outr):
        sv = s2r[0] + s2r[1]
        tv = td2r[0, :, 0:_DE] + td2r[1, :, 0:_DE]
        dv = td2r[0, :, _DE:_DE + 1] + td2r[1, :, _DE:_DE + 1]
        sums = (jnp.dot(sv, wmar[...], preferred_element_type=jnp.float32)
                + jnp.dot(tv, wmbr[...], preferred_element_type=jnp.float32)
                + dv * bmr[...])
        hn = sums / jnp.maximum(dv, 1.0)
        out = (jnp.dot(nfr[...], wa1r[...], preferred_element_type=jnp.float32)
               + jnp.dot(hn, wa2r[...], preferred_element_type=jnp.float32)
               + bar[...])
        outr[...] = jnp.maximum(out, 0.0)

    return pl.pallas_call(
        body,
        grid=grid,
        in_specs=[
            pl.BlockSpec((_NC, bn, _DIN), lambda i: (0, i, 0)),
            pl.BlockSpec((_NC, bn, _DTD), lambda i: (0, i, 0)),
            pl.BlockSpec((bn, _DIN), lambda i: (i, 0)),
            pl.BlockSpec((_DIN, _DOUT), lambda i: (0, 0)),
            pl.BlockSpec((_DE, _DOUT), lambda i: (0, 0)),
            pl.BlockSpec((1, _DOUT), lambda i: (0, 0)),
            pl.BlockSpec((_DIN, _DOUT), lambda i: (0, 0)),
            pl.BlockSpec((_DOUT, _DOUT), lambda i: (0, 0)),
            pl.BlockSpec((1, _DOUT), lambda i: (0, 0)),
        ],
        out_specs=pl.BlockSpec((bn, _DOUT), lambda i: (i, 0)),
        out_shape=jax.ShapeDtypeStruct((_N, _DOUT), jnp.float32),
    )(s2, td2, nf, wma, wmb, bm, wa1, wa2, ba)


def kernel(nfeats, efeats, edge_index, W_msg, b_msg, W_apply, b_apply):
    nf = nfeats.reshape(_N, _DIN)
    ef = efeats.reshape(_E, _DE)
    src3 = edge_index[0].reshape(_NW, _C, _K)
    dst3 = edge_index[1].reshape(_NW, _C, _K)
    s2, td2 = _sc_segment_sums(nf, ef, src3, dst3)
    wma = W_msg[:, :_DIN].T
    wmb = W_msg[:, _DIN:].T
    wa1 = W_apply[:, :_DIN].T
    wa2 = W_apply[:, _DIN:].T
    out = _tc_apply(s2, td2, nf, wma, wmb,
                    b_msg.reshape(1, _DOUT), wa1, wa2,
                    b_apply.reshape(1, _DOUT))
    return out.reshape(_N, 1, _DOUT)


# speedup vs baseline: 1.3499x; 1.0327x over previous
import functools

import jax
import jax.numpy as jnp
from jax import lax
from jax.experimental import pallas as pl
from jax.experimental.pallas import tpu as pltpu
from jax.experimental.pallas import tpu_sc as plsc

_N = 10000
_E = 320000
_DIN = 128
_DE = 16
_DOUT = 128
_DTD = 24

_NC = 2
_NS = 16
_NW = _NC * _NS
_EPW = _E // _NW
_K = 80
_C = _EPW // _K
_NP = 10240
_RPT = _NP // _NS
_G = 25
_NG = _C // _G


def _sc_segment_sums(nf, ef, src3, dst3):
    mesh = plsc.VectorSubcoreMesh(core_axis_name="c", subcore_axis_name="s")

    @functools.partial(
        pl.kernel,
        mesh=mesh,
        out_type=[
            jax.ShapeDtypeStruct((_NC, _NP, _DIN), jnp.float32),
            jax.ShapeDtypeStruct((_NC, _NP, _DTD), jnp.float32),
        ],
        scratch_types=[
            pltpu.VMEM_SHARED((_NP, _DIN), jnp.float32),
            pltpu.VMEM_SHARED((_NP, _DTD), jnp.float32),
            pltpu.VMEM((_G, _K), jnp.int32),
            pltpu.VMEM((_G, _K), jnp.int32),
            pltpu.VMEM((2, _K, _DIN), jnp.float32),
            pltpu.VMEM((2, _K, _DTD), jnp.float32),
            pltpu.SemaphoreType.DMA,
            pltpu.SemaphoreType.DMA,
            pltpu.SemaphoreType.DMA,
            pltpu.SemaphoreType.DMA,
        ],
        compiler_params=pltpu.CompilerParams(use_tc_tiling_on_sc=False),
    )
    def k(nf_hbm, ef_hbm, src_hbm, dst_hbm, s_out, td_out,
          s_sp, td_sp, isrc, idst, rows, tdv, sem_g, sem_e, sem_s, sem_td):
        c = lax.axis_index("c")
        s = lax.axis_index("s")
        w = c * _NS + s
        base = s * _RPT
        ebase = w * _EPW

        def zrow(i, carry):
            for h in range(_DIN // 16):
                rows[0, i, pl.ds(h * 16, 16)] = jnp.zeros((16,), jnp.float32)
            tdv[0, i, pl.ds(0, 16)] = jnp.zeros((16,), jnp.float32)
            tdv[0, i, pl.ds(8, 16)] = jnp.zeros((16,), jnp.float32)
            return carry
        lax.fori_loop(0, _K, zrow, 0)

        for z in range(_RPT // _K):
            r0 = base + z * _K
            pltpu.sync_copy(rows.at[0], s_sp.at[pl.ds(r0, _K)])
            pltpu.sync_copy(tdv.at[0], td_sp.at[pl.ds(r0, _K)])

        def orow(i, carry):
            tdv[0, i, pl.ds(8, 16)] = jnp.ones((16,), jnp.float32)
            tdv[1, i, pl.ds(8, 16)] = jnp.ones((16,), jnp.float32)
            return carry
        lax.fori_loop(0, _K, orow, 0)
        plsc.subcore_barrier()

        def stage(g, carry):
            pltpu.sync_copy(src_hbm.at[w, pl.ds(g * _G, _G)], isrc)
            pltpu.sync_copy(dst_hbm.at[w, pl.ds(g * _G, _G)], idst)
            pltpu.async_copy(nf_hbm.at[isrc.at[0]], rows.at[0], sem_g)
            pltpu.async_copy(ef_hbm.at[pl.ds(ebase + g * _G * _K, _K)],
                             tdv.at[0, :, pl.ds(0, _DE)], sem_e)

            def step(j, carry2):
                p = lax.rem(j, 2)
                pltpu.make_async_copy(
                    nf_hbm.at[pl.ds(0, _K)], rows.at[p], sem_g).wait()
                pltpu.make_async_copy(
                    ef_hbm.at[pl.ds(0, _K)],
                    tdv.at[p, :, pl.ds(0, _DE)], sem_e).wait()

                @pl.when(j >= 1)
                def _drain_prev():
                    pltpu.make_async_copy(
                        nf_hbm.at[pl.ds(0, _K)], rows.at[1 - p], sem_s).wait()
                    pltpu.make_async_copy(
                        td_out.at[0, pl.ds(0, _K)], tdv.at[1 - p],
                        sem_td).wait()

                @pl.when(j + 1 < _G)
                def _prefetch():
                    pltpu.async_copy(
                        nf_hbm.at[isrc.at[j + 1]], rows.at[1 - p], sem_g)
                    pltpu.async_copy(
                        ef_hbm.at[pl.ds(ebase + (g * _G + j + 1) * _K, _K)],
                        tdv.at[1 - p, :, pl.ds(0, _DE)], sem_e)

                pltpu.async_copy(rows.at[p], s_sp.at[idst.at[j]], sem_s,
                                 add=True)
                pltpu.async_copy(tdv.at[p], td_sp.at[idst.at[j]], sem_td,
                                 add=True)
                return carry2
            lax.fori_loop(0, _G, step, 0)

            pltpu.make_async_copy(
                nf_hbm.at[pl.ds(0, _K)], rows.at[0], sem_s).wait()
            pltpu.make_async_copy(
                td_out.at[0, pl.ds(0, _K)], tdv.at[0], sem_td).wait()
            return carry
        lax.fori_loop(0, _NG, stage, 0)

        plsc.subcore_barrier()

        pltpu.sync_copy(s_sp.at[pl.ds(base, _RPT)],
                        s_out.at[c, pl.ds(base, _RPT)])
        pltpu.sync_copy(td_sp.at[pl.ds(base, _RPT)],
                        td_out.at[c, pl.ds(base, _RPT)])

    return k(nf, ef, src3, dst3)


def _tc_apply(s2, td2, nf, wma, wmb, bm, wa1, wa2, ba):
    bn = 2000
    grid = (_N // bn,)

    def body(s2r, td2r, nfr, wmar, wmbr, bmr, wa1r, wa2r, bar, outr):
        sv = s2r[0] + s2r[1]
        tv = td2r[0, :, 0:_DE] + td2r[1, :, 0:_DE]
        dv = td2r[0, :, _DE:_DE + 1] + td2r[1, :, _DE:_DE + 1]
        sums = (jnp.dot(sv, wmar[...], preferred_element_type=jnp.float32)
                + jnp.dot(tv, wmbr[...], preferred_element_type=jnp.float32)
                + dv * bmr[...])
        hn = sums / jnp.maximum(dv, 1.0)
        out = (jnp.dot(nfr[...], wa1r[...], preferred_element_type=jnp.float32)
               + jnp.dot(hn, wa2r[...], preferred_element_type=jnp.float32)
               + bar[...])
        outr[...] = jnp.maximum(out, 0.0)

    return pl.pallas_call(
        body,
        grid=grid,
        in_specs=[
            pl.BlockSpec((_NC, bn, _DIN), lambda i: (0, i, 0)),
            pl.BlockSpec((_NC, bn, _DTD), lambda i: (0, i, 0)),
            pl.BlockSpec((bn, _DIN), lambda i: (i, 0)),
            pl.BlockSpec((_DIN, _DOUT), lambda i: (0, 0)),
            pl.BlockSpec((_DE, _DOUT), lambda i: (0, 0)),
            pl.BlockSpec((1, _DOUT), lambda i: (0, 0)),
            pl.BlockSpec((_DIN, _DOUT), lambda i: (0, 0)),
            pl.BlockSpec((_DOUT, _DOUT), lambda i: (0, 0)),
            pl.BlockSpec((1, _DOUT), lambda i: (0, 0)),
        ],
        out_specs=pl.BlockSpec((bn, _DOUT), lambda i: (i, 0)),
        out_shape=jax.ShapeDtypeStruct((_N, _DOUT), jnp.float32),
    )(s2, td2, nf, wma, wmb, bm, wa1, wa2, ba)


def kernel(nfeats, efeats, edge_index, W_msg, b_msg, W_apply, b_apply):
    nf = nfeats.reshape(_N, _DIN)
    ef = efeats.reshape(_E, _DE)
    src3 = edge_index[0].reshape(_NW, _C, _K)
    dst3 = edge_index[1].reshape(_NW, _C, _K)
    s2, td2 = _sc_segment_sums(nf, ef, src3, dst3)
    wma = W_msg[:, :_DIN].T
    wmb = W_msg[:, _DIN:].T
    wa1 = W_apply[:, :_DIN].T
    wa2 = W_apply[:, _DIN:].T
    out = _tc_apply(s2, td2, nf, wma, wmb,
                    b_msg.reshape(1, _DOUT), wa1, wa2,
                    b_apply.reshape(1, _DOUT))
    return out.reshape(_N, 1, _DOUT)
